# baseline (speedup 1.0000x reference)
import functools

import jax
import jax.numpy as jnp
import numpy as np
from jax import lax
from jax.experimental import pallas as pl
from jax.experimental.pallas import tpu as pltpu
from jax.experimental.pallas import tpu_sc as plsc

D = 128
B = 96
NBUF = 3
NC = 2
NS = 16
NW = NC * NS


def _sc_scatter_gather(n_nodes, n_edges, nblk):
  p_rows = -(-(n_nodes + 1) // (8 * NS)) * (8 * NS)
  rows_per_tile = p_rows // NS
  epw = nblk * B
  e = n_edges
  w_str = e // epw
  assert w_str >= 1 and (e - w_str * epw) % 8 == 0 and e % 8 == 0

  mesh = plsc.VectorSubcoreMesh(core_axis_name="c", subcore_axis_name="s")

  @functools.partial(
      pl.kernel,
      mesh=mesh,
      compiler_params=pltpu.CompilerParams(use_tc_tiling_on_sc=False),
      out_type=jax.ShapeDtypeStruct((NC, p_rows, D), jnp.float32),
      scratch_types=[
          pltpu.VMEM_SHARED((p_rows, D), jnp.float32),
          pltpu.VMEM((epw,), jnp.int32),
      ] + [pltpu.VMEM((B,), jnp.int32) for _ in range(NBUF)]
        + [pltpu.VMEM((B, D), jnp.float32) for _ in range(NBUF)]
        + [pltpu.SemaphoreType.DMA] * (3 * NBUF),
  )
  def k(x_hbm, edge_hbm, rowpad_hbm, colpad_hbm, zero_hbm, out_hbm,
        accum, colb, *bufs_and_sems):
    rslots = bufs_and_sems[:NBUF]
    bufs = bufs_and_sems[NBUF:2 * NBUF]
    gsems = bufs_and_sems[2 * NBUF:3 * NBUF]
    rsems = bufs_and_sems[3 * NBUF:4 * NBUF]
    ssems = bufs_and_sems[4 * NBUF:5 * NBUF]

    c = lax.axis_index("c")
    s = lax.axis_index("s")
    wid = c * NS + s
    e0 = wid * epw

    @pl.when(wid < w_str)
    def _():
      pltpu.sync_copy(edge_hbm.at[1, pl.ds(e0, epw)], colb)

    for w in range(w_str, NW):
      @pl.when(wid == w)
      def _(w=w):
        ms = min(max(e - w * epw, 0), epw)
        if ms:
          pltpu.sync_copy(edge_hbm.at[1, pl.ds(w * epw, ms)],
                          colb.at[pl.ds(0, ms)])
        po = w * epw + ms - e
        pltpu.sync_copy(colpad_hbm.at[pl.ds(po, epw - ms)],
                        colb.at[pl.ds(ms, epw - ms)])

    def stage_rows(g, slot, sem):
      start = e0 + g * B
      rem = e % B

      @pl.when(start + B <= e)
      def _():
        pltpu.async_copy(edge_hbm.at[0, pl.ds(start, B)], slot, sem)

      @pl.when(start >= e)
      def _():
        pltpu.async_copy(rowpad_hbm.at[pl.ds(start - e, B)], slot, sem)

      if rem:
        @pl.when(jnp.logical_and(start < e, start + B > e))
        def _():
          pltpu.async_copy(edge_hbm.at[0, pl.ds(e - rem, rem)],
                           slot.at[pl.ds(0, rem)], sem)
          pltpu.async_copy(rowpad_hbm.at[pl.ds(0, B - rem)],
                           slot.at[pl.ds(rem, B - rem)], sem)

    for b in range(NBUF):
      stage_rows(b, rslots[b], rsems[b])
      pltpu.async_copy(x_hbm.at[colb.at[pl.ds(b * B, B)]], bufs[b], gsems[b])

    r0 = s * rows_per_tile
    pltpu.sync_copy(zero_hbm, accum.at[pl.ds(r0, rows_per_tile)])

    plsc.subcore_barrier()

    def body(i, carry):
      for b in range(NBUF):
        g = i * NBUF + b
        bp = (b - 1) % NBUF
        pltpu.make_async_copy(
            x_hbm.at[colb.at[pl.ds(g * B, B)]], bufs[b], gsems[b]).wait()
        pltpu.make_async_copy(
            rowpad_hbm.at[pl.ds(0, B)], rslots[b], rsems[b]).wait()
        pltpu.async_copy(bufs[b], accum.at[rslots[b]], ssems[b], add=True)

        @pl.when(jnp.logical_and(g >= 1, g + 2 < nblk))
        def _():
          g2 = g + 2
          pltpu.make_async_copy(
              bufs[bp], accum.at[rslots[bp]], ssems[bp]).wait()
          stage_rows(g2, rslots[bp], rsems[bp])
          pltpu.async_copy(
              x_hbm.at[colb.at[pl.ds(g2 * B, B)]], bufs[bp], gsems[bp])
      return carry

    lax.fori_loop(0, nblk // NBUF, body, 0, unroll=False)

    for b in range(NBUF):
      pltpu.make_async_copy(bufs[b], accum.at[rslots[b]], ssems[b]).wait()

    plsc.subcore_barrier()

    pltpu.sync_copy(accum.at[pl.ds(r0, rows_per_tile)],
                    out_hbm.at[c, pl.ds(r0, rows_per_tile)])

  return k, p_rows


def _tc_combine(partials, n_nodes):
  blk = 2000

  def add_k(p_ref, o_ref):
    o_ref[...] = p_ref[0] + p_ref[1]

  return pl.pallas_call(
      add_k,
      grid=(n_nodes // blk,),
      in_specs=[pl.BlockSpec((2, blk, D), lambda i: (0, i, 0))],
      out_specs=pl.BlockSpec((blk, D), lambda i: (i, 0)),
      out_shape=jax.ShapeDtypeStruct((n_nodes, D), jnp.float32),
  )(partials)


@jax.jit
def kernel(x, edge_index):
  n_nodes = x.shape[0]
  e = edge_index.shape[1]
  edge32 = edge_index.astype(jnp.int32)

  nblk = -(-e // (NW * B))
  nblk = -(-nblk // NBUF) * NBUF
  e_pad = NW * nblk * B
  pad = e_pad - e
  p_rows = -(-(n_nodes + 1) // (8 * NS)) * (8 * NS)
  pad_idx = np.arange(max(pad, 1), dtype=np.int32)
  row_pad = jnp.asarray(n_nodes + pad_idx % (p_rows - n_nodes))
  col_pad = jnp.asarray(pad_idx % n_nodes)

  sc_k, p_rows2 = _sc_scatter_gather(n_nodes, e, nblk)
  assert p_rows2 == p_rows
  zeros = jnp.zeros((p_rows // NS, D), jnp.float32)
  partials = sc_k(x, edge32, row_pad, col_pad, zeros)
  return _tc_combine(partials, n_nodes)


# speedup vs baseline: 1.0623x; 1.0623x over previous
import functools

import jax
import jax.numpy as jnp
import numpy as np
from jax import lax
from jax.experimental import pallas as pl
from jax.experimental.pallas import tpu as pltpu
from jax.experimental.pallas import tpu_sc as plsc

D = 128
B = 72
NBUF = 4
NC = 2
NS = 16
NW = NC * NS


def _sc_scatter_gather(n_nodes, n_edges, nblk):
  p_rows = -(-(n_nodes + 1) // (8 * NS)) * (8 * NS)
  rows_per_tile = p_rows // NS
  epw = nblk * B
  e = n_edges
  w_str = e // epw
  assert w_str >= 1 and (e - w_str * epw) % 8 == 0 and e % 8 == 0

  mesh = plsc.VectorSubcoreMesh(core_axis_name="c", subcore_axis_name="s")

  @functools.partial(
      pl.kernel,
      mesh=mesh,
      compiler_params=pltpu.CompilerParams(use_tc_tiling_on_sc=False),
      out_type=jax.ShapeDtypeStruct((NC, p_rows, D), jnp.float32),
      scratch_types=[
          pltpu.VMEM_SHARED((p_rows, D), jnp.float32),
          pltpu.VMEM((epw,), jnp.int32),
      ] + [pltpu.VMEM((B,), jnp.int32) for _ in range(NBUF)]
        + [pltpu.VMEM((B, D), jnp.float32) for _ in range(NBUF)]
        + [pltpu.SemaphoreType.DMA] * (2 * NBUF),
  )
  def k(x_hbm, edge_hbm, rowpad_hbm, colpad_hbm, zero_hbm, out_hbm,
        accum, colb, *bufs_and_sems):
    rslots = bufs_and_sems[:NBUF]
    bufs = bufs_and_sems[NBUF:2 * NBUF]
    gsems = bufs_and_sems[2 * NBUF:3 * NBUF]
    rsems = bufs_and_sems[3 * NBUF:4 * NBUF]

    c = lax.axis_index("c")
    s = lax.axis_index("s")
    wid = c * NS + s
    e0 = wid * epw

    @pl.when(wid < w_str)
    def _():
      pltpu.sync_copy(edge_hbm.at[1, pl.ds(e0, epw)], colb)

    for w in range(w_str, NW):
      @pl.when(wid == w)
      def _(w=w):
        ms = min(max(e - w * epw, 0), epw)
        if ms:
          pltpu.sync_copy(edge_hbm.at[1, pl.ds(w * epw, ms)],
                          colb.at[pl.ds(0, ms)])
        po = w * epw + ms - e
        pltpu.sync_copy(colpad_hbm.at[pl.ds(po, epw - ms)],
                        colb.at[pl.ds(ms, epw - ms)])

    def stage_rows(g, slot, sem):
      start = e0 + g * B
      rem = e % B

      @pl.when(start + B <= e)
      def _():
        pltpu.async_copy(edge_hbm.at[0, pl.ds(start, B)], slot, sem)

      @pl.when(start >= e)
      def _():
        pltpu.async_copy(rowpad_hbm.at[pl.ds(start - e, B)], slot, sem)

      if rem:
        @pl.when(jnp.logical_and(start < e, start + B > e))
        def _():
          pltpu.async_copy(edge_hbm.at[0, pl.ds(e - rem, rem)],
                           slot.at[pl.ds(0, rem)], sem)
          pltpu.async_copy(rowpad_hbm.at[pl.ds(0, B - rem)],
                           slot.at[pl.ds(rem, B - rem)], sem)

    for b in range(NBUF):
      stage_rows(b, rslots[b], rsems[b])
      pltpu.async_copy(x_hbm.at[colb.at[pl.ds(b * B, B)]], bufs[b], gsems[b])

    r0 = s * rows_per_tile
    pltpu.sync_copy(zero_hbm, accum.at[pl.ds(r0, rows_per_tile)])

    plsc.subcore_barrier()

    def body(i, carry):
      for b in range(NBUF):
        g = i * NBUF + b
        pltpu.make_async_copy(
            x_hbm.at[colb.at[pl.ds(g * B, B)]], bufs[b], gsems[b]).wait()
        pltpu.make_async_copy(
            rowpad_hbm.at[pl.ds(0, B)], rslots[b], rsems[b]).wait()
        pltpu.sync_copy(bufs[b], accum.at[rslots[b]], add=True)

        @pl.when(g + NBUF < nblk)
        def _():
          g2 = g + NBUF
          stage_rows(g2, rslots[b], rsems[b])
          pltpu.async_copy(
              x_hbm.at[colb.at[pl.ds(g2 * B, B)]], bufs[b], gsems[b])
      return carry

    lax.fori_loop(0, nblk // NBUF, body, 0, unroll=False)

    plsc.subcore_barrier()

    pltpu.sync_copy(accum.at[pl.ds(r0, rows_per_tile)],
                    out_hbm.at[c, pl.ds(r0, rows_per_tile)])

  return k, p_rows


def _tc_combine(partials, n_nodes):
  blk = 2000

  def add_k(p_ref, o_ref):
    o_ref[...] = p_ref[0] + p_ref[1]

  return pl.pallas_call(
      add_k,
      grid=(n_nodes // blk,),
      in_specs=[pl.BlockSpec((2, blk, D), lambda i: (0, i, 0))],
      out_specs=pl.BlockSpec((blk, D), lambda i: (i, 0)),
      out_shape=jax.ShapeDtypeStruct((n_nodes, D), jnp.float32),
  )(partials)


@jax.jit
def kernel(x, edge_index):
  n_nodes = x.shape[0]
  e = edge_index.shape[1]
  edge32 = edge_index.astype(jnp.int32)

  nblk = -(-e // (NW * B))
  nblk = -(-nblk // NBUF) * NBUF
  e_pad = NW * nblk * B
  pad = e_pad - e
  p_rows = -(-(n_nodes + 1) // (8 * NS)) * (8 * NS)
  pad_idx = np.arange(max(pad, 1), dtype=np.int32)
  row_pad = jnp.asarray(n_nodes + pad_idx % (p_rows - n_nodes))
  col_pad = jnp.asarray(pad_idx % n_nodes)

  sc_k, p_rows2 = _sc_scatter_gather(n_nodes, e, nblk)
  assert p_rows2 == p_rows
  zeros = jnp.zeros((p_rows // NS, D), jnp.float32)
  partials = sc_k(x, edge32, row_pad, col_pad, zeros)
  return _tc_combine(partials, n_nodes)


# speedup vs baseline: 1.0643x; 1.0018x over previous
import functools

import jax
import jax.numpy as jnp
import numpy as np
from jax import lax
from jax.experimental import pallas as pl
from jax.experimental.pallas import tpu as pltpu
from jax.experimental.pallas import tpu_sc as plsc

D = 128
B = 56
NBUF = 5
NC = 2
NS = 16
NW = NC * NS


def _sc_scatter_gather(n_nodes, n_edges, nblk):
  p_rows = -(-(n_nodes + 1) // (8 * NS)) * (8 * NS)
  rows_per_tile = p_rows // NS
  epw = nblk * B
  e = n_edges
  w_str = e // epw
  assert w_str >= 1 and (e - w_str * epw) % 8 == 0 and e % 8 == 0

  mesh = plsc.VectorSubcoreMesh(core_axis_name="c", subcore_axis_name="s")

  @functools.partial(
      pl.kernel,
      mesh=mesh,
      compiler_params=pltpu.CompilerParams(use_tc_tiling_on_sc=False),
      out_type=jax.ShapeDtypeStruct((NC, p_rows, D), jnp.float32),
      scratch_types=[
          pltpu.VMEM_SHARED((p_rows, D), jnp.float32),
          pltpu.VMEM((epw,), jnp.int32),
      ] + [pltpu.VMEM((B,), jnp.int32) for _ in range(NBUF)]
        + [pltpu.VMEM((B, D), jnp.float32) for _ in range(NBUF)]
        + [pltpu.SemaphoreType.DMA] * (2 * NBUF),
  )
  def k(x_hbm, edge_hbm, rowpad_hbm, colpad_hbm, zero_hbm, out_hbm,
        accum, colb, *bufs_and_sems):
    rslots = bufs_and_sems[:NBUF]
    bufs = bufs_and_sems[NBUF:2 * NBUF]
    gsems = bufs_and_sems[2 * NBUF:3 * NBUF]
    rsems = bufs_and_sems[3 * NBUF:4 * NBUF]

    c = lax.axis_index("c")
    s = lax.axis_index("s")
    wid = c * NS + s
    e0 = wid * epw

    @pl.when(wid < w_str)
    def _():
      pltpu.sync_copy(edge_hbm.at[1, pl.ds(e0, epw)], colb)

    for w in range(w_str, NW):
      @pl.when(wid == w)
      def _(w=w):
        ms = min(max(e - w * epw, 0), epw)
        if ms:
          pltpu.sync_copy(edge_hbm.at[1, pl.ds(w * epw, ms)],
                          colb.at[pl.ds(0, ms)])
        po = w * epw + ms - e
        pltpu.sync_copy(colpad_hbm.at[pl.ds(po, epw - ms)],
                        colb.at[pl.ds(ms, epw - ms)])

    def stage_rows(g, slot, sem):
      start = e0 + g * B
      rem = e % B

      @pl.when(start + B <= e)
      def _():
        pltpu.async_copy(edge_hbm.at[0, pl.ds(start, B)], slot, sem)

      @pl.when(start >= e)
      def _():
        pltpu.async_copy(rowpad_hbm.at[pl.ds(start - e, B)], slot, sem)

      if rem:
        @pl.when(jnp.logical_and(start < e, start + B > e))
        def _():
          pltpu.async_copy(edge_hbm.at[0, pl.ds(e - rem, rem)],
                           slot.at[pl.ds(0, rem)], sem)
          pltpu.async_copy(rowpad_hbm.at[pl.ds(0, B - rem)],
                           slot.at[pl.ds(rem, B - rem)], sem)

    for b in range(NBUF):
      stage_rows(b, rslots[b], rsems[b])
      pltpu.async_copy(x_hbm.at[colb.at[pl.ds(b * B, B)]], bufs[b], gsems[b])

    r0 = s * rows_per_tile
    pltpu.sync_copy(zero_hbm, accum.at[pl.ds(r0, rows_per_tile)])

    plsc.subcore_barrier()

    def body(i, carry):
      for b in range(NBUF):
        g = i * NBUF + b
        pltpu.make_async_copy(
            x_hbm.at[colb.at[pl.ds(g * B, B)]], bufs[b], gsems[b]).wait()
        pltpu.make_async_copy(
            rowpad_hbm.at[pl.ds(0, B)], rslots[b], rsems[b]).wait()
        pltpu.sync_copy(bufs[b], accum.at[rslots[b]], add=True)

        @pl.when(g + NBUF < nblk)
        def _():
          g2 = g + NBUF
          stage_rows(g2, rslots[b], rsems[b])
          pltpu.async_copy(
              x_hbm.at[colb.at[pl.ds(g2 * B, B)]], bufs[b], gsems[b])
      return carry

    lax.fori_loop(0, nblk // NBUF, body, 0, unroll=False)

    plsc.subcore_barrier()

    pltpu.sync_copy(accum.at[pl.ds(r0, rows_per_tile)],
                    out_hbm.at[c, pl.ds(r0, rows_per_tile)])

  return k, p_rows


def _tc_combine(partials, n_nodes):
  blk = 2000

  def add_k(p_ref, o_ref):
    o_ref[...] = p_ref[0] + p_ref[1]

  return pl.pallas_call(
      add_k,
      grid=(n_nodes // blk,),
      in_specs=[pl.BlockSpec((2, blk, D), lambda i: (0, i, 0))],
      out_specs=pl.BlockSpec((blk, D), lambda i: (i, 0)),
      out_shape=jax.ShapeDtypeStruct((n_nodes, D), jnp.float32),
  )(partials)


@jax.jit
def kernel(x, edge_index):
  n_nodes = x.shape[0]
  e = edge_index.shape[1]
  edge32 = edge_index.astype(jnp.int32)

  nblk = -(-e // (NW * B))
  nblk = -(-nblk // NBUF) * NBUF
  e_pad = NW * nblk * B
  pad = e_pad - e
  p_rows = -(-(n_nodes + 1) // (8 * NS)) * (8 * NS)
  pad_idx = np.arange(max(pad, 1), dtype=np.int32)
  row_pad = jnp.asarray(n_nodes + pad_idx % (p_rows - n_nodes))
  col_pad = jnp.asarray(pad_idx % n_nodes)

  sc_k, p_rows2 = _sc_scatter_gather(n_nodes, e, nblk)
  assert p_rows2 == p_rows
  zeros = jnp.zeros((p_rows // NS, D), jnp.float32)
  partials = sc_k(x, edge32, row_pad, col_pad, zeros)
  return _tc_combine(partials, n_nodes)
